# baseline (speedup 1.0000x reference)
import functools

import numpy as np
import jax
import jax.numpy as jnp
from jax import lax
from jax.experimental import pallas as pl
from jax.experimental.pallas import tpu as pltpu
from jax.experimental.pallas import tpu_sc as plsc

N_NODES = 10000
N_EDGES = 160000
D_IN = 128
D_OUT = 15

NC = 2
NS = 16
NW = NC * NS
CHUNK = 125
EH = N_EDGES // 2
EWH = EH // NW
CHH = EWH // CHUNK
EW = N_EDGES // NW
CH = EW // CHUNK
RPT = N_NODES // NS
BPAIR = 1600

_q = np.arange(N_EDGES)
_i, _rem = _q // (2 * BPAIR), _q % (2 * BPAIR)
_r, _j = _rem // 8, _rem % 8
_EDGE_OF_SLOT = (2 * (_i * BPAIR + _r + (_j // 2) * (BPAIR // 4)) + (_j % 2))

_U_SEL = [0, 1, 2, 3, 4, 4, 4, 5, 5, 5, 6, 6, 6, 6, 6]
_S_SEL = [0, 0, 0, 0, 1, 2, 3, 1, 2, 3, 4, 5, 6, 7, 8]


def _sc_gather(nf, srcp):
    mesh = plsc.VectorSubcoreMesh(core_axis_name="c", subcore_axis_name="s")

    @functools.partial(
        pl.kernel,
        mesh=mesh,
        out_type=(
            jax.ShapeDtypeStruct((EH, D_IN), jnp.float32),
            jax.ShapeDtypeStruct((EH, D_IN), jnp.float32),
        ),
        scratch_types=[
            pltpu.VMEM((2 * CHH, CHUNK), jnp.int32),
            pltpu.VMEM((CHUNK, D_IN), jnp.float32),
            pltpu.VMEM((CHUNK, D_IN), jnp.float32),
            pltpu.SemaphoreType.DMA,
            pltpu.SemaphoreType.DMA,
        ],
        compiler_params=pltpu.CompilerParams(use_tc_tiling_on_sc=False),
    )
    def k(nf_hbm, src_hbm, oute_hbm, outo_hbm, idx_v, buf0, buf1, sem0, sem1):
        c = lax.axis_index("c")
        s = lax.axis_index("s")
        wid = s * NC + c
        pltpu.sync_copy(src_hbm.at[0, pl.ds(wid * CHH, CHH)],
                        idx_v.at[pl.ds(0, CHH)])
        pltpu.sync_copy(src_hbm.at[1, pl.ds(wid * CHH, CHH)],
                        idx_v.at[pl.ds(CHH, CHH)])
        bufs = (buf0, buf1)
        sems = (sem0, sem1)
        pltpu.async_copy(nf_hbm.at[idx_v.at[0]], buf0, sem0)

        def body(j, carry):
            slot = lax.rem(j, 2)
            parity = j // CHH
            jj = lax.rem(j, CHH)

            def step(b, sm, other_b, other_sm):
                @pl.when(j + 1 < 2 * CHH)
                def _start():
                    pltpu.async_copy(nf_hbm.at[idx_v.at[j + 1]], other_b,
                                     other_sm)

                pltpu.make_async_copy(nf_hbm.at[idx_v.at[j]], b, sm).wait()
                row = wid * EWH + jj * CHUNK

                @pl.when(parity == 0)
                def _we():
                    pltpu.sync_copy(b, oute_hbm.at[pl.ds(row, CHUNK)])

                @pl.when(parity == 1)
                def _wo():
                    pltpu.sync_copy(b, outo_hbm.at[pl.ds(row, CHUNK)])

            @pl.when(slot == 0)
            def _even():
                step(bufs[0], sems[0], bufs[1], sems[1])

            @pl.when(slot == 1)
            def _odd():
                step(bufs[1], sems[1], bufs[0], sems[0])

            return carry

        lax.fori_loop(0, 2 * CHH, body, 0)

    return k(nf, srcp)


def _sc_scatter(msg, dst2d, zmat):
    mesh = plsc.VectorSubcoreMesh(core_axis_name="c", subcore_axis_name="s")

    @functools.partial(
        pl.kernel,
        mesh=mesh,
        out_type=jax.ShapeDtypeStruct((NC, N_NODES, 16), jnp.float32),
        scratch_types=[
            pltpu.VMEM((CH, CHUNK), jnp.int32),
            pltpu.VMEM((EW, 16), jnp.float32),
            pltpu.VMEM((RPT, 16), jnp.float32),
            pltpu.VMEM_SHARED((N_NODES, 16), jnp.float32),
        ],
        compiler_params=pltpu.CompilerParams(use_tc_tiling_on_sc=False),
    )
    def k(msg_hbm, dst_hbm, z_hbm, out_hbm, idx_v, msg_v, bnc, acc):
        c = lax.axis_index("c")
        s = lax.axis_index("s")
        wid = s * NC + c
        pltpu.sync_copy(z_hbm.at[pl.ds(s * RPT, RPT)], bnc)
        pltpu.sync_copy(bnc, acc.at[pl.ds(s * RPT, RPT)])
        pltpu.sync_copy(dst_hbm.at[pl.ds(wid * CH, CH)], idx_v)
        pltpu.sync_copy(msg_hbm.at[pl.ds(wid * EW, EW)], msg_v)
        plsc.subcore_barrier()

        def body(j, carry):
            pltpu.sync_copy(
                msg_v.at[pl.ds(j * CHUNK, CHUNK)], acc.at[idx_v.at[j]], add=True
            )
            return carry

        lax.fori_loop(0, CH, body, 0)
        plsc.subcore_barrier()
        pltpu.sync_copy(acc.at[pl.ds(s * RPT, RPT)], bnc)
        pltpu.sync_copy(bnc, out_hbm.at[c, pl.ds(s * RPT, RPT)])

    return k(msg, dst2d, zmat)


def _tc_dense(g_even, g_odd, radial2, sh8, W1blk, W2blk, A, B8):
    BP = BPAIR

    def body(ge_ref, go_ref, r_ref, sh_ref, w1_ref, w2_ref, a_ref, b_ref,
             o_ref):
        r = r_ref[...]
        h1 = jnp.dot(r, w1_ref[...], preferred_element_type=jnp.float32)
        h = h1 / (1.0 + jnp.exp(-h1))
        w = jnp.dot(h.astype(jnp.bfloat16), w2_ref[...],
                    preferred_element_type=jnp.float32)
        ge = ge_ref[...]
        go = go_ref[...]
        acc_e = jnp.zeros((BP, 16), jnp.float32)
        acc_o = jnp.zeros((BP, 16), jnp.float32)
        for u in range(7):
            re = jnp.sum(w[:, u * 128:(u + 1) * 128] * ge, axis=1,
                         keepdims=True)
            ro = jnp.sum(w[:, 896 + u * 128:896 + (u + 1) * 128] * go, axis=1,
                         keepdims=True)
            a_row = a_ref[u, :][None, :]
            acc_e = acc_e + re * a_row
            acc_o = acc_o + ro * a_row
        acc32 = jnp.concatenate([acc_e, acc_o], axis=1)
        q = BP // 4
        acc = jnp.concatenate(
            [acc32[k * q:(k + 1) * q] for k in range(4)], axis=1)
        se = jnp.dot(sh_ref[...], b_ref[...],
                     preferred_element_type=jnp.float32)
        o_ref[...] = acc * se

    return pl.pallas_call(
        body,
        grid=(EH // BP,),
        in_specs=[
            pl.BlockSpec((BP, 128), lambda i: (i, 0)),
            pl.BlockSpec((BP, 128), lambda i: (i, 0)),
            pl.BlockSpec((BP, 128), lambda i: (i, 0)),
            pl.BlockSpec((BP // 4, 128), lambda i: (i, 0)),
            pl.BlockSpec((128, 128), lambda i: (0, 0)),
            pl.BlockSpec((128, 1792), lambda i: (0, 0)),
            pl.BlockSpec((8, 16), lambda i: (0, 0)),
            pl.BlockSpec((128, 128), lambda i: (0, 0)),
        ],
        out_specs=pl.BlockSpec((BP // 4, 128), lambda i: (i, 0)),
        out_shape=jax.ShapeDtypeStruct((N_EDGES // 8, 128), jnp.float32),
    )(g_even, g_odd, radial2, sh8, W1blk, W2blk, A, B8)


def _tc_final(partials, nf, wselfp):
    def body(p_ref, nf_ref, ws_ref, o_ref):
        s0 = jnp.dot(nf_ref[...], ws_ref[...], preferred_element_type=jnp.float32)
        o_ref[...] = p_ref[0] + p_ref[1] + s0

    return pl.pallas_call(
        body,
        out_shape=jax.ShapeDtypeStruct((N_NODES, 16), jnp.float32),
    )(partials, nf, wselfp)


def kernel(node_features, edge_index, edge_sh, edge_radial, W1, W2, W_self):
    src = edge_index[0]
    dst = edge_index[1]
    srcp = jnp.stack([src[0::2], src[1::2]]).reshape(2, EH // CHUNK, CHUNK)
    eq = jnp.asarray(_EDGE_OF_SLOT, dtype=jnp.int32)
    dst2d = dst[eq].reshape(N_EDGES // CHUNK, CHUNK)

    radial2 = edge_radial.reshape(EH, 128)
    sh8 = jnp.pad(edge_sh, ((0, 0), (0, 16 - 9)))[eq].reshape(
        N_EDGES // 8, 128)

    W1s = W1 * (1.0 / np.sqrt(64.0))
    W1blk = jnp.zeros((128, 128), jnp.float32)
    W1blk = W1blk.at[:64, :64].set(W1s).at[64:, 64:].set(W1s)

    w2_scale = 1.0 / (np.sqrt(64.0) * np.sqrt(float(D_IN)) * 4.0)
    W2q = (W2.reshape(64, D_IN, 7).transpose(0, 2, 1).reshape(64, 7 * D_IN)
           * w2_scale).astype(jnp.bfloat16)
    W2blk = jnp.zeros((128, 1792), jnp.bfloat16)
    W2blk = W2blk.at[:64, :896].set(W2q).at[64:, 896:].set(W2q)

    A = np.zeros((8, 16), np.float32)
    B16 = np.zeros((16, 16), np.float32)
    for o in range(D_OUT):
        A[_U_SEL[o], o] = 1.0
        B16[_S_SEL[o], o] = 1.0
    A = jnp.asarray(A)
    B8 = jnp.asarray(np.kron(np.eye(8, dtype=np.float32), B16))

    wselfp = jnp.pad(W_self, ((0, 0), (0, 16 - 4))) * (1.0 / np.sqrt(float(D_IN)))
    zmat = jnp.zeros((N_NODES, 16), jnp.float32)

    g_even, g_odd = _sc_gather(node_features, srcp)
    msg8 = _tc_dense(g_even, g_odd, radial2, sh8, W1blk, W2blk, A, B8)
    msg = msg8.reshape(N_EDGES, 16)
    partials = _sc_scatter(msg, dst2d, zmat)
    out16 = _tc_final(partials, node_features, wselfp)
    return out16[:, :D_OUT]


# speedup vs baseline: 1.3914x; 1.3914x over previous
import functools

import numpy as np
import jax
import jax.numpy as jnp
from jax import lax
from jax.experimental import pallas as pl
from jax.experimental.pallas import tpu as pltpu
from jax.experimental.pallas import tpu_sc as plsc

N_NODES = 10000
N_EDGES = 160000
D_IN = 128
D_OUT = 15

NC = 2
NS = 16
NW = NC * NS
CHUNK = 125
EW = N_EDGES // NW
CH = EW // CHUNK
RPT = N_NODES // NS
BE = 3200
QB = BE // 8

_q = np.arange(N_EDGES)
_EDGE_OF_SLOT = ((_q // 8 // QB) * BE + (_q % 8) * QB + (_q // 8) % QB)

_U_SEL = [0, 1, 2, 3, 4, 4, 4, 5, 5, 5, 6, 6, 6, 6, 6]
_S_SEL = [0, 0, 0, 0, 1, 2, 3, 1, 2, 3, 4, 5, 6, 7, 8]


def _sc_gather(nf, src2d):
    mesh = plsc.VectorSubcoreMesh(core_axis_name="c", subcore_axis_name="s")

    @functools.partial(
        pl.kernel,
        mesh=mesh,
        out_type=jax.ShapeDtypeStruct((N_EDGES, D_IN), jnp.float32),
        scratch_types=[
            pltpu.VMEM((CH, CHUNK), jnp.int32),
            pltpu.VMEM((CHUNK, D_IN), jnp.float32),
            pltpu.VMEM((CHUNK, D_IN), jnp.float32),
            pltpu.SemaphoreType.DMA,
            pltpu.SemaphoreType.DMA,
        ],
        compiler_params=pltpu.CompilerParams(use_tc_tiling_on_sc=False),
    )
    def k(nf_hbm, src_hbm, out_hbm, idx_v, buf0, buf1, sem0, sem1):
        c = lax.axis_index("c")
        s = lax.axis_index("s")
        wid = s * NC + c
        pltpu.sync_copy(src_hbm.at[pl.ds(wid * CH, CH)], idx_v)
        bufs = (buf0, buf1)
        sems = (sem0, sem1)
        pltpu.async_copy(nf_hbm.at[idx_v.at[0]], buf0, sem0)

        def body(j, carry):
            slot = lax.rem(j, 2)

            def step(b, sm, other_b, other_sm):
                @pl.when(j + 1 < CH)
                def _start():
                    pltpu.async_copy(nf_hbm.at[idx_v.at[j + 1]], other_b, other_sm)

                pltpu.make_async_copy(nf_hbm.at[idx_v.at[j]], b, sm).wait()
                pltpu.sync_copy(b, out_hbm.at[pl.ds(wid * EW + j * CHUNK, CHUNK)])

            @pl.when(slot == 0)
            def _even():
                step(bufs[0], sems[0], bufs[1], sems[1])

            @pl.when(slot == 1)
            def _odd():
                step(bufs[1], sems[1], bufs[0], sems[0])

            return carry

        lax.fori_loop(0, CH, body, 0)

    return k(nf, src2d)


def _sc_scatter(msg, dst2d, zmat):
    mesh = plsc.VectorSubcoreMesh(core_axis_name="c", subcore_axis_name="s")

    @functools.partial(
        pl.kernel,
        mesh=mesh,
        out_type=jax.ShapeDtypeStruct((NC, N_NODES, 16), jnp.float32),
        scratch_types=[
            pltpu.VMEM((CH, CHUNK), jnp.int32),
            pltpu.VMEM((EW, 16), jnp.float32),
            pltpu.VMEM((RPT, 16), jnp.float32),
            pltpu.VMEM_SHARED((N_NODES, 16), jnp.float32),
        ],
        compiler_params=pltpu.CompilerParams(use_tc_tiling_on_sc=False),
    )
    def k(msg_hbm, dst_hbm, z_hbm, out_hbm, idx_v, msg_v, bnc, acc):
        c = lax.axis_index("c")
        s = lax.axis_index("s")
        wid = s * NC + c
        pltpu.sync_copy(z_hbm.at[pl.ds(s * RPT, RPT)], bnc)
        pltpu.sync_copy(bnc, acc.at[pl.ds(s * RPT, RPT)])
        pltpu.sync_copy(dst_hbm.at[pl.ds(wid * CH, CH)], idx_v)
        pltpu.sync_copy(msg_hbm.at[pl.ds(wid * EW, EW)], msg_v)
        plsc.subcore_barrier()

        def body(j, carry):
            pltpu.sync_copy(
                msg_v.at[pl.ds(j * CHUNK, CHUNK)], acc.at[idx_v.at[j]], add=True
            )
            return carry

        lax.fori_loop(0, CH, body, 0)
        plsc.subcore_barrier()
        pltpu.sync_copy(acc.at[pl.ds(s * RPT, RPT)], bnc)
        pltpu.sync_copy(bnc, out_hbm.at[c, pl.ds(s * RPT, RPT)])

    return k(msg, dst2d, zmat)


def _tc_dense(gathered, radial, shp, W1s, W2q, A, B):
    def body(g_ref, r_ref, sh_ref, w1_ref, w2_ref, a_ref, b_ref, o_ref):
        r = r_ref[...]
        h1 = jnp.dot(r, w1_ref[...], preferred_element_type=jnp.float32)
        h = h1 / (1.0 + jnp.exp(-h1))
        w = jnp.dot(h.astype(jnp.bfloat16), w2_ref[...],
                    preferred_element_type=jnp.float32)
        g = g_ref[...]
        acc = jnp.zeros((BE, 16), jnp.float32)
        for u in range(7):
            red = jnp.sum(w[:, u * D_IN:(u + 1) * D_IN] * g, axis=1,
                          keepdims=True)
            acc = acc + red * a_ref[u, :][None, :]
        se = jnp.dot(sh_ref[...], b_ref[...], preferred_element_type=jnp.float32)
        msg = acc * se
        o_ref[...] = jnp.concatenate(
            [msg[k * QB:(k + 1) * QB] for k in range(8)], axis=1)

    return pl.pallas_call(
        body,
        grid=(N_EDGES // BE,),
        in_specs=[
            pl.BlockSpec((BE, D_IN), lambda i: (i, 0)),
            pl.BlockSpec((BE, 64), lambda i: (i, 0)),
            pl.BlockSpec((BE, 9), lambda i: (i, 0)),
            pl.BlockSpec((64, 64), lambda i: (0, 0)),
            pl.BlockSpec((64, 896), lambda i: (0, 0)),
            pl.BlockSpec((8, 16), lambda i: (0, 0)),
            pl.BlockSpec((9, 16), lambda i: (0, 0)),
        ],
        out_specs=pl.BlockSpec((QB, 128), lambda i: (i, 0)),
        out_shape=jax.ShapeDtypeStruct((N_EDGES // 8, 128), jnp.float32),
    )(gathered, radial, shp, W1s, W2q, A, B)


def _tc_final(partials, nf, wselfp):
    def body(p_ref, nf_ref, ws_ref, o_ref):
        s0 = jnp.dot(nf_ref[...], ws_ref[...], preferred_element_type=jnp.float32)
        o_ref[...] = p_ref[0] + p_ref[1] + s0

    return pl.pallas_call(
        body,
        out_shape=jax.ShapeDtypeStruct((N_NODES, 16), jnp.float32),
    )(partials, nf, wselfp)


def kernel(node_features, edge_index, edge_sh, edge_radial, W1, W2, W_self):
    src2d = edge_index[0].reshape(N_EDGES // CHUNK, CHUNK)
    eq = jnp.asarray(_EDGE_OF_SLOT, dtype=jnp.int32)
    dst2d = edge_index[1][eq].reshape(N_EDGES // CHUNK, CHUNK)

    W1s = W1 * (1.0 / np.sqrt(64.0))
    w2_scale = 1.0 / (np.sqrt(64.0) * np.sqrt(float(D_IN)) * 4.0)
    W2q = (W2.reshape(64, D_IN, 7).transpose(0, 2, 1).reshape(64, 7 * D_IN)
           * w2_scale).astype(jnp.bfloat16)

    A = np.zeros((8, 16), np.float32)
    B = np.zeros((16, 16), np.float32)
    for o in range(D_OUT):
        A[_U_SEL[o], o] = 1.0
        B[_S_SEL[o], o] = 1.0
    A = jnp.asarray(A)
    B9 = jnp.asarray(B[:9])

    wselfp = jnp.pad(W_self, ((0, 0), (0, 16 - 4))) * (1.0 / np.sqrt(float(D_IN)))
    zmat = jnp.zeros((N_NODES, 16), jnp.float32)

    gathered = _sc_gather(node_features, src2d)
    msg8 = _tc_dense(gathered, edge_radial, edge_sh, W1s, W2q, A, B9)
    msg = msg8.reshape(N_EDGES, 16)
    partials = _sc_scatter(msg, dst2d, zmat)
    out16 = _tc_final(partials, node_features, wselfp)
    return out16[:, :D_OUT]


# speedup vs baseline: 1.4340x; 1.0306x over previous
import functools

import numpy as np
import jax
import jax.numpy as jnp
from jax import lax
from jax.experimental import pallas as pl
from jax.experimental.pallas import tpu as pltpu
from jax.experimental.pallas import tpu_sc as plsc

N_NODES = 10000
N_EDGES = 160000
D_IN = 128
D_OUT = 15

NC = 2
NS = 16
NW = NC * NS
CHUNK = 125
EW = N_EDGES // NW
CH = EW // CHUNK
RPT = N_NODES // NS
BE = 3200
QB = BE // 8

_q = np.arange(N_EDGES)
_EDGE_OF_SLOT = ((_q // 8 // QB) * BE + (_q % 8) * QB + (_q // 8) % QB)

_U_SEL = [0, 1, 2, 3, 4, 4, 4, 5, 5, 5, 6, 6, 6, 6, 6]
_S_SEL = [0, 0, 0, 0, 1, 2, 3, 1, 2, 3, 4, 5, 6, 7, 8]


def _sc_gather(nf, src2d):
    mesh = plsc.VectorSubcoreMesh(core_axis_name="c", subcore_axis_name="s")

    @functools.partial(
        pl.kernel,
        mesh=mesh,
        out_type=jax.ShapeDtypeStruct((N_EDGES, D_IN), jnp.float32),
        scratch_types=[
            pltpu.VMEM((CH, CHUNK), jnp.int32),
            pltpu.VMEM((CHUNK, D_IN), jnp.float32),
            pltpu.VMEM((CHUNK, D_IN), jnp.float32),
            pltpu.SemaphoreType.DMA,
            pltpu.SemaphoreType.DMA,
        ],
        compiler_params=pltpu.CompilerParams(use_tc_tiling_on_sc=False),
    )
    def k(nf_hbm, src_hbm, out_hbm, idx_v, buf0, buf1, sem0, sem1):
        c = lax.axis_index("c")
        s = lax.axis_index("s")
        wid = s * NC + c
        pltpu.sync_copy(src_hbm.at[pl.ds(wid * CH, CH)], idx_v)
        bufs = (buf0, buf1)
        sems = (sem0, sem1)
        pltpu.async_copy(nf_hbm.at[idx_v.at[0]], buf0, sem0)

        def body(j, carry):
            slot = lax.rem(j, 2)

            def step(b, sm, other_b, other_sm):
                @pl.when(j + 1 < CH)
                def _start():
                    pltpu.async_copy(nf_hbm.at[idx_v.at[j + 1]], other_b, other_sm)

                pltpu.make_async_copy(nf_hbm.at[idx_v.at[j]], b, sm).wait()
                pltpu.sync_copy(b, out_hbm.at[pl.ds(wid * EW + j * CHUNK, CHUNK)])

            @pl.when(slot == 0)
            def _even():
                step(bufs[0], sems[0], bufs[1], sems[1])

            @pl.when(slot == 1)
            def _odd():
                step(bufs[1], sems[1], bufs[0], sems[0])

            return carry

        lax.fori_loop(0, CH, body, 0)

    return k(nf, src2d)


def _sc_scatter(msg, dst2d, zmat):
    mesh = plsc.VectorSubcoreMesh(core_axis_name="c", subcore_axis_name="s")

    @functools.partial(
        pl.kernel,
        mesh=mesh,
        out_type=jax.ShapeDtypeStruct((NC, N_NODES, 16), jnp.float32),
        scratch_types=[
            pltpu.VMEM((CH, CHUNK), jnp.int32),
            pltpu.VMEM((EW, 16), jnp.float32),
            pltpu.VMEM((RPT, 16), jnp.float32),
            pltpu.VMEM_SHARED((N_NODES, 16), jnp.float32),
        ],
        compiler_params=pltpu.CompilerParams(use_tc_tiling_on_sc=False),
    )
    def k(msg_hbm, dst_hbm, z_hbm, out_hbm, idx_v, msg_v, bnc, acc):
        c = lax.axis_index("c")
        s = lax.axis_index("s")
        wid = s * NC + c
        pltpu.sync_copy(z_hbm.at[pl.ds(s * RPT, RPT)], bnc)
        pltpu.sync_copy(bnc, acc.at[pl.ds(s * RPT, RPT)])
        pltpu.sync_copy(dst_hbm.at[pl.ds(wid * CH, CH)], idx_v)
        pltpu.sync_copy(msg_hbm.at[pl.ds(wid * EW, EW)], msg_v)
        plsc.subcore_barrier()

        def body(j, carry):
            pltpu.sync_copy(
                msg_v.at[pl.ds(j * CHUNK, CHUNK)], acc.at[idx_v.at[j]], add=True
            )
            return carry

        lax.fori_loop(0, CH, body, 0)
        plsc.subcore_barrier()
        pltpu.sync_copy(acc.at[pl.ds(s * RPT, RPT)], bnc)
        pltpu.sync_copy(bnc, out_hbm.at[c, pl.ds(s * RPT, RPT)])

    return k(msg, dst2d, zmat)


def _tc_dense(gathered, radial, shp, W1s, W2q, A, B):
    def body(g_ref, r_ref, sh_ref, w1_ref, w2_ref, a_ref, b_ref, o_ref):
        r = r_ref[...]
        h1 = jnp.dot(r, w1_ref[...], preferred_element_type=jnp.float32)
        h = h1 / (1.0 + jnp.exp(-h1))
        w = jnp.dot(h.astype(jnp.bfloat16), w2_ref[...],
                    preferred_element_type=jnp.float32)
        g = g_ref[...]
        cols = []
        for u in range(7):
            cols.append(jnp.sum(w[:, u * D_IN:(u + 1) * D_IN] * g, axis=1,
                                keepdims=True))
        cols.append(jnp.zeros_like(cols[0]))
        proj = jnp.concatenate(cols, axis=1)
        pe = jnp.dot(proj, a_ref[...], preferred_element_type=jnp.float32)
        se = jnp.dot(sh_ref[...], b_ref[...], preferred_element_type=jnp.float32)
        msg = pe * se
        o_ref[...] = jnp.concatenate(
            [msg[k * QB:(k + 1) * QB] for k in range(8)], axis=1)

    return pl.pallas_call(
        body,
        grid=(N_EDGES // BE,),
        in_specs=[
            pl.BlockSpec((BE, D_IN), lambda i: (i, 0)),
            pl.BlockSpec((BE, 64), lambda i: (i, 0)),
            pl.BlockSpec((BE, 9), lambda i: (i, 0)),
            pl.BlockSpec((64, 64), lambda i: (0, 0)),
            pl.BlockSpec((64, 896), lambda i: (0, 0)),
            pl.BlockSpec((8, 16), lambda i: (0, 0)),
            pl.BlockSpec((9, 16), lambda i: (0, 0)),
        ],
        out_specs=pl.BlockSpec((QB, 128), lambda i: (i, 0)),
        out_shape=jax.ShapeDtypeStruct((N_EDGES // 8, 128), jnp.float32),
    )(gathered, radial, shp, W1s, W2q, A, B)


def _tc_final(partials, nf, wselfp):
    def body(p_ref, nf_ref, ws_ref, o_ref):
        s0 = jnp.dot(nf_ref[...], ws_ref[...], preferred_element_type=jnp.float32)
        o_ref[...] = p_ref[0] + p_ref[1] + s0

    return pl.pallas_call(
        body,
        out_shape=jax.ShapeDtypeStruct((N_NODES, 16), jnp.float32),
    )(partials, nf, wselfp)


def kernel(node_features, edge_index, edge_sh, edge_radial, W1, W2, W_self):
    src2d = edge_index[0].reshape(N_EDGES // CHUNK, CHUNK)
    eq = jnp.asarray(_EDGE_OF_SLOT, dtype=jnp.int32)
    dst2d = edge_index[1][eq].reshape(N_EDGES // CHUNK, CHUNK)

    W1s = W1 * (1.0 / np.sqrt(64.0))
    w2_scale = 1.0 / (np.sqrt(64.0) * np.sqrt(float(D_IN)) * 4.0)
    W2q = (W2.reshape(64, D_IN, 7).transpose(0, 2, 1).reshape(64, 7 * D_IN)
           * w2_scale).astype(jnp.bfloat16)

    A = np.zeros((8, 16), np.float32)
    B = np.zeros((16, 16), np.float32)
    for o in range(D_OUT):
        A[_U_SEL[o], o] = 1.0
        B[_S_SEL[o], o] = 1.0
    A = jnp.asarray(A)
    B9 = jnp.asarray(B[:9])

    wselfp = jnp.pad(W_self, ((0, 0), (0, 16 - 4))) * (1.0 / np.sqrt(float(D_IN)))
    zmat = jnp.zeros((N_NODES, 16), jnp.float32)

    gathered = _sc_gather(node_features, src2d)
    msg8 = _tc_dense(gathered, edge_radial, edge_sh, W1s, W2q, A, B9)
    msg = msg8.reshape(N_EDGES, 16)
    partials = _sc_scatter(msg, dst2d, zmat)
    out16 = _tc_final(partials, node_features, wselfp)
    return out16[:, :D_OUT]


# speedup vs baseline: 1.4651x; 1.0217x over previous
import functools

import numpy as np
import jax
import jax.numpy as jnp
from jax import lax
from jax.experimental import pallas as pl
from jax.experimental.pallas import tpu as pltpu
from jax.experimental.pallas import tpu_sc as plsc

N_NODES = 10000
N_EDGES = 160000
D_IN = 128
D_OUT = 15

NC = 2
NS = 16
NW = NC * NS
CHUNK = 125
EW = N_EDGES // NW
CH = EW // CHUNK
RPT = N_NODES // NS
BE = 3200
QB = BE // 8

_q = np.arange(N_EDGES)
_EDGE_OF_SLOT = ((_q // 8 // QB) * BE + (_q % 8) * QB + (_q // 8) % QB)

_U_SEL = [0, 1, 2, 3, 4, 4, 4, 5, 5, 5, 6, 6, 6, 6, 6]
_S_SEL = [0, 0, 0, 0, 1, 2, 3, 1, 2, 3, 4, 5, 6, 7, 8]


def _sc_gather(nf, src2d):
    mesh = plsc.VectorSubcoreMesh(core_axis_name="c", subcore_axis_name="s")

    @functools.partial(
        pl.kernel,
        mesh=mesh,
        out_type=jax.ShapeDtypeStruct((N_EDGES, D_IN), jnp.float32),
        scratch_types=[
            pltpu.VMEM((CH, CHUNK), jnp.int32),
            pltpu.VMEM((CHUNK, D_IN), jnp.float32),
            pltpu.VMEM((CHUNK, D_IN), jnp.float32),
            pltpu.SemaphoreType.DMA,
            pltpu.SemaphoreType.DMA,
        ],
        compiler_params=pltpu.CompilerParams(use_tc_tiling_on_sc=False),
    )
    def k(nf_hbm, src_hbm, out_hbm, idx_v, buf0, buf1, sem0, sem1):
        c = lax.axis_index("c")
        s = lax.axis_index("s")
        wid = s * NC + c
        pltpu.sync_copy(src_hbm.at[pl.ds(wid * CH, CH)], idx_v)
        bufs = (buf0, buf1)
        sems = (sem0, sem1)
        pltpu.async_copy(nf_hbm.at[idx_v.at[0]], buf0, sem0)

        def body(j, carry):
            slot = lax.rem(j, 2)

            def step(b, sm, other_b, other_sm):
                @pl.when(j + 1 < CH)
                def _start():
                    pltpu.async_copy(nf_hbm.at[idx_v.at[j + 1]], other_b, other_sm)

                pltpu.make_async_copy(nf_hbm.at[idx_v.at[j]], b, sm).wait()
                pltpu.sync_copy(b, out_hbm.at[pl.ds(wid * EW + j * CHUNK, CHUNK)])

            @pl.when(slot == 0)
            def _even():
                step(bufs[0], sems[0], bufs[1], sems[1])

            @pl.when(slot == 1)
            def _odd():
                step(bufs[1], sems[1], bufs[0], sems[0])

            return carry

        lax.fori_loop(0, CH, body, 0)

    return k(nf, src2d)


def _sc_scatter(msg, dst2d, zmat):
    mesh = plsc.VectorSubcoreMesh(core_axis_name="c", subcore_axis_name="s")

    @functools.partial(
        pl.kernel,
        mesh=mesh,
        out_type=jax.ShapeDtypeStruct((NC, N_NODES, 16), jnp.float32),
        scratch_types=[
            pltpu.VMEM((CH, CHUNK), jnp.int32),
            pltpu.VMEM((EW, 16), jnp.float32),
            pltpu.VMEM((RPT, 16), jnp.float32),
            pltpu.VMEM_SHARED((N_NODES, 16), jnp.float32),
        ],
        compiler_params=pltpu.CompilerParams(use_tc_tiling_on_sc=False),
    )
    def k(msg_hbm, dst_hbm, z_hbm, out_hbm, idx_v, msg_v, bnc, acc):
        c = lax.axis_index("c")
        s = lax.axis_index("s")
        wid = s * NC + c
        pltpu.sync_copy(z_hbm.at[pl.ds(s * RPT, RPT)], bnc)
        pltpu.sync_copy(bnc, acc.at[pl.ds(s * RPT, RPT)])
        pltpu.sync_copy(dst_hbm.at[pl.ds(wid * CH, CH)], idx_v)
        pltpu.sync_copy(msg_hbm.at[pl.ds(wid * EW, EW)], msg_v)
        plsc.subcore_barrier()

        def body(j, carry):
            pltpu.sync_copy(
                msg_v.at[pl.ds(j * CHUNK, CHUNK)], acc.at[idx_v.at[j]], add=True
            )
            return carry

        lax.fori_loop(0, CH, body, 0)
        plsc.subcore_barrier()
        pltpu.sync_copy(acc.at[pl.ds(s * RPT, RPT)], bnc)
        pltpu.sync_copy(bnc, out_hbm.at[c, pl.ds(s * RPT, RPT)])

    return k(msg, dst2d, zmat)


def _tc_dense(gathered, radial, shp, W1s, W2q, A, B):
    def body(g_ref, r_ref, sh_ref, w1_ref, w2_ref, a_ref, b_ref, o_ref):
        r = r_ref[...]
        h1 = jnp.dot(r, w1_ref[...], preferred_element_type=jnp.float32)
        h = h1 / (1.0 + jnp.exp(-h1))
        w = jnp.dot(h.astype(jnp.bfloat16), w2_ref[...],
                    preferred_element_type=jnp.float32)
        g = g_ref[...]
        cols = []
        for u in range(7):
            cols.append(jnp.sum(w[:, u * D_IN:(u + 1) * D_IN] * g, axis=1,
                                keepdims=True))
        cols.append(jnp.zeros_like(cols[0]))
        proj = jnp.concatenate(cols, axis=1)
        pe = jnp.dot(proj, a_ref[...], preferred_element_type=jnp.float32)
        se = jnp.dot(sh_ref[...], b_ref[...], preferred_element_type=jnp.float32)
        msg = pe * se
        o_ref[...] = jnp.concatenate(
            [msg[k * QB:(k + 1) * QB] for k in range(8)], axis=1)

    return pl.pallas_call(
        body,
        grid=(N_EDGES // BE,),
        in_specs=[
            pl.BlockSpec((BE, D_IN), lambda i: (i, 0)),
            pl.BlockSpec((BE, 64), lambda i: (i, 0)),
            pl.BlockSpec((BE, 9), lambda i: (i, 0)),
            pl.BlockSpec((64, 64), lambda i: (0, 0)),
            pl.BlockSpec((64, 896), lambda i: (0, 0)),
            pl.BlockSpec((8, 16), lambda i: (0, 0)),
            pl.BlockSpec((9, 16), lambda i: (0, 0)),
        ],
        out_specs=pl.BlockSpec((QB, 128), lambda i: (i, 0)),
        out_shape=jax.ShapeDtypeStruct((N_EDGES // 8, 128), jnp.float32),
    )(gathered, radial, shp, W1s, W2q, A, B)


def _tc_final(partials, nf, wselfp):
    def body(p_ref, nf_ref, ws_ref, o_ref):
        s0 = jnp.dot(nf_ref[...], ws_ref[...], preferred_element_type=jnp.float32)
        o_ref[...] = p_ref[0] + p_ref[1] + s0

    return pl.pallas_call(
        body,
        out_shape=jax.ShapeDtypeStruct((N_NODES, 16), jnp.float32),
    )(partials, nf, wselfp)


def kernel(node_features, edge_index, edge_sh, edge_radial, W1, W2, W_self):
    src2d = edge_index[0].reshape(N_EDGES // CHUNK, CHUNK)
    eq = jnp.asarray(_EDGE_OF_SLOT, dtype=jnp.int32)
    dst2d = edge_index[1][eq].reshape(N_EDGES // CHUNK, CHUNK)

    W1s = (W1 * (1.0 / np.sqrt(64.0))).astype(jnp.bfloat16)
    w2_scale = 1.0 / (np.sqrt(64.0) * np.sqrt(float(D_IN)) * 4.0)
    W2q = (W2.reshape(64, D_IN, 7).transpose(0, 2, 1).reshape(64, 7 * D_IN)
           * w2_scale).astype(jnp.bfloat16)

    A = np.zeros((8, 16), np.float32)
    B = np.zeros((16, 16), np.float32)
    for o in range(D_OUT):
        A[_U_SEL[o], o] = 1.0
        B[_S_SEL[o], o] = 1.0
    A = jnp.asarray(A)
    B9 = jnp.asarray(B[:9]).astype(jnp.bfloat16)

    wselfp = jnp.pad(W_self, ((0, 0), (0, 16 - 4))) * (1.0 / np.sqrt(float(D_IN)))
    zmat = jnp.zeros((N_NODES, 16), jnp.float32)

    gathered = _sc_gather(node_features, src2d)
    msg8 = _tc_dense(gathered, edge_radial.astype(jnp.bfloat16),
                     edge_sh.astype(jnp.bfloat16), W1s, W2q, A, B9)
    msg = msg8.reshape(N_EDGES, 16)
    partials = _sc_scatter(msg, dst2d, zmat)
    out16 = _tc_final(partials, node_features, wselfp)
    return out16[:, :D_OUT]
